# Initial kernel scaffold; baseline (speedup 1.0000x reference)
import jax
import jax.numpy as jnp
from jax.experimental import pallas as pl


def kernel(X, ppr_scores, ppr_idx, W1, W2):
    raise NotImplementedError("write your pallas kernel here")


# speedup vs baseline: 2.4245x; 2.4245x over previous
import functools

import jax
import jax.numpy as jnp
from jax import lax
from jax.experimental import pallas as pl
from jax.experimental.pallas import tpu as pltpu

_BIG = 1 << 30


def _fused_body(x_ref, s_ref, idx_ref, w1_ref, w2_ref, out_ref, acc_ref, *,
                n_seg, win):
    i = pl.program_id(0)

    @pl.when(i == 0)
    def _init():
        acc_ref[...] = jnp.zeros_like(acc_ref)

    h = jnp.maximum(
        jnp.dot(x_ref[...], w1_ref[...], preferred_element_type=jnp.float32),
        0.0)
    logits = jnp.dot(h, w2_ref[...], preferred_element_type=jnp.float32)
    weighted = logits * s_ref[...]
    idx_row = idx_ref[...]
    r = idx_row.shape[1]

    start0 = (jnp.min(idx_row) // 8) * 8

    def cond(c):
        return c[0] < _BIG

    def wbody(c):
        start = c[0]
        rel = idx_row - start
        iota = lax.broadcasted_iota(jnp.int32, (win, r), 0)
        onehot = (rel == iota).astype(jnp.float32)
        contrib = jnp.dot(onehot, weighted,
                          preferred_element_type=jnp.float32)
        acc_ref[pl.ds(start, win), :] += contrib
        nxt = jnp.min(jnp.where(rel >= win, idx_row, _BIG))
        return ((nxt // 8) * 8,)

    lax.while_loop(cond, wbody, (start0,))

    @pl.when(i == pl.num_programs(0) - 1)
    def _flush():
        out_ref[...] = acc_ref[0:n_seg, :]


def kernel(X, ppr_scores, ppr_idx, W1, W2):
    n, f = X.shape
    h_dim = W1.shape[1]
    c = W2.shape[1]
    b = 10000
    win = 256
    r_blk = 1280
    assert n % r_blk == 0
    grid = n // r_blk

    scores2d = ppr_scores.reshape(n, 1)
    idx2d = ppr_idx.reshape(1, n)

    body = functools.partial(_fused_body, n_seg=b, win=win)
    out = pl.pallas_call(
        body,
        grid=(grid,),
        in_specs=[
            pl.BlockSpec((r_blk, f), lambda i: (i, 0)),
            pl.BlockSpec((r_blk, 1), lambda i: (i, 0)),
            pl.BlockSpec((1, r_blk), lambda i: (0, i)),
            pl.BlockSpec((f, h_dim), lambda i: (0, 0)),
            pl.BlockSpec((h_dim, c), lambda i: (0, 0)),
        ],
        out_specs=pl.BlockSpec((b, c), lambda i: (0, 0)),
        out_shape=jax.ShapeDtypeStruct((b, c), jnp.float32),
        scratch_shapes=[pltpu.VMEM((b + win, c), jnp.float32)],
    )(X, scores2d, idx2d, W1, W2)
    return out


# speedup vs baseline: 2.4357x; 1.0046x over previous
import functools

import jax
import jax.numpy as jnp
from jax import lax
from jax.experimental import pallas as pl
from jax.experimental.pallas import tpu as pltpu

_BIG = 1 << 30


def _fused_body(x_ref, s_ref, idx_ref, w1_ref, w2_ref, out_ref, acc_ref, *,
                n_seg, win):
    i = pl.program_id(0)

    @pl.when(i == 0)
    def _init():
        acc_ref[...] = jnp.zeros_like(acc_ref)

    h = jnp.maximum(
        jnp.dot(x_ref[...].astype(jnp.bfloat16), w1_ref[...],
                preferred_element_type=jnp.float32),
        0.0)
    logits = jnp.dot(h.astype(jnp.bfloat16), w2_ref[...],
                     preferred_element_type=jnp.float32)
    weighted = (logits * s_ref[...]).astype(jnp.bfloat16)
    idx_row = idx_ref[...]
    r = idx_row.shape[1]

    start0 = (jnp.min(idx_row) // 8) * 8

    def cond(c):
        return c[0] < _BIG

    def wbody(c):
        start = c[0]
        rel = idx_row - start
        iota = lax.broadcasted_iota(jnp.int32, (win, r), 0)
        onehot = (rel == iota).astype(jnp.bfloat16)
        contrib = jnp.dot(onehot, weighted,
                          preferred_element_type=jnp.float32)
        acc_ref[pl.ds(start, win), :] += contrib
        nxt = jnp.min(jnp.where(rel >= win, idx_row, _BIG))
        return ((nxt // 8) * 8,)

    lax.while_loop(cond, wbody, (start0,))

    @pl.when(i == pl.num_programs(0) - 1)
    def _flush():
        out_ref[...] = acc_ref[0:n_seg, :]


def kernel(X, ppr_scores, ppr_idx, W1, W2):
    n, f = X.shape
    h_dim = W1.shape[1]
    c = W2.shape[1]
    b = 10000
    win = 256
    r_blk = 1280
    assert n % r_blk == 0
    grid = n // r_blk

    scores2d = ppr_scores.reshape(n, 1)
    idx2d = ppr_idx.reshape(1, n)

    body = functools.partial(_fused_body, n_seg=b, win=win)
    out = pl.pallas_call(
        body,
        grid=(grid,),
        in_specs=[
            pl.BlockSpec((r_blk, f), lambda i: (i, 0)),
            pl.BlockSpec((r_blk, 1), lambda i: (i, 0)),
            pl.BlockSpec((1, r_blk), lambda i: (0, i)),
            pl.BlockSpec((f, h_dim), lambda i: (0, 0)),
            pl.BlockSpec((h_dim, c), lambda i: (0, 0)),
        ],
        out_specs=pl.BlockSpec((b, c), lambda i: (0, 0)),
        out_shape=jax.ShapeDtypeStruct((b, c), jnp.float32),
        scratch_shapes=[pltpu.VMEM((b + win, c), jnp.float32)],
    )(X, scores2d, idx2d, W1.astype(jnp.bfloat16), W2.astype(jnp.bfloat16))
    return out
